# baseline (speedup 1.0000x reference)
import functools

import jax
import jax.numpy as jnp
from jax import lax
from jax.experimental import pallas as pl
from jax.experimental.pallas import tpu as pltpu
from jax.experimental.pallas import tpu_sc as plsc

NVD = 1024
NPX = 1024
NPY = 1024
B = NPX * NPY
NC = 2
NS = 16
L = 16
NW = NC * NS
PTS_PER_W = B // NW
VECS = PTS_PER_W // L
MAX_DEPTH = 16
VERT_BIT = 1 << 20
UNROLL = 8


def _prep_body(lm_ref, rm_ref, sd_ref, packed_ref):
    col = lax.broadcasted_iota(jnp.int32, (NVD, NVD), 1).astype(jnp.float32)
    li = jnp.sum(lm_ref[...] * col, axis=1, keepdims=True).astype(jnp.int32)
    ri = jnp.sum(rm_ref[...] * col, axis=1, keepdims=True).astype(jnp.int32)
    vert = (sd_ref[...] != 0.0).astype(jnp.int32)
    packed_ref[...] = (vert << 20) | (li << 10) | ri


def _pack_tables(left_matrix, right_matrix, split_directions):
    packed = pl.pallas_call(
        _prep_body,
        out_shape=jax.ShapeDtypeStruct((NVD, 1), jnp.int32),
    )(left_matrix, right_matrix, split_directions.reshape(NVD, 1))
    return packed.reshape(NVD)


_mesh = plsc.VectorSubcoreMesh(core_axis_name="c", subcore_axis_name="s")


@functools.partial(
    pl.kernel,
    mesh=_mesh,
    out_type=jax.ShapeDtypeStruct((B,), jnp.float32),
    compiler_params=pltpu.CompilerParams(needs_layout_passes=False),
    scratch_types=[
        pltpu.VMEM((PTS_PER_W,), jnp.float32),
        pltpu.VMEM((PTS_PER_W,), jnp.float32),
        pltpu.VMEM((NVD,), jnp.int32),
        pltpu.VMEM((NVD,), jnp.float32),
        pltpu.VMEM((NVD,), jnp.float32),
        pltpu.VMEM((PTS_PER_W,), jnp.float32),
        pltpu.VMEM((L,), jnp.int32),
    ],
)
def _traverse(xs_hbm, ys_hbm, packed_hbm, sp_hbm, vals_hbm, zero_hbm, out_hbm,
              xs_v, ys_v, packed_v, sp_v, vals_v, out_v, zero_v):
    wid = lax.axis_index("s") * NC + lax.axis_index("c")
    base = wid * PTS_PER_W
    pltpu.sync_copy(xs_hbm.at[pl.ds(base, PTS_PER_W)], xs_v)
    pltpu.sync_copy(ys_hbm.at[pl.ds(base, PTS_PER_W)], ys_v)
    pltpu.sync_copy(packed_hbm, packed_v)
    pltpu.sync_copy(sp_hbm, sp_v)
    pltpu.sync_copy(vals_hbm, vals_v)
    pltpu.sync_copy(zero_hbm, zero_v)

    idx0 = zero_v[...]

    def body(i, carry):
        pbase0 = i * (L * UNROLL)
        x = [None] * UNROLL
        y = [None] * UNROLL
        idx = [idx0] * UNROLL
        minx = [jnp.zeros((L,), jnp.float32)] * UNROLL
        maxx = [jnp.ones((L,), jnp.float32)] * UNROLL
        miny = [jnp.zeros((L,), jnp.float32)] * UNROLL
        maxy = [jnp.ones((L,), jnp.float32)] * UNROLL
        for u in range(UNROLL):
            pb = pbase0 + u * L
            x[u] = xs_v[pl.ds(pb, L)]
            y[u] = ys_v[pl.ds(pb, L)]
        for _ in range(MAX_DEPTH):
            for u in range(UNROLL):
                packed = plsc.load_gather(packed_v, [idx[u]])
                sp = plsc.load_gather(sp_v, [idx[u]])
                horiz = packed < VERT_BIT
                lchild = (packed >> 10) & (NVD - 1)
                rchild = packed & (NVD - 1)
                mn = jnp.where(horiz, minx[u], miny[u])
                mx = jnp.where(horiz, maxx[u], maxy[u])
                pos = jnp.where(horiz, x[u], y[u])
                split = mn + sp * (mx - mn)
                is_left = pos < split
                nmn = jnp.where(is_left, mn, split)
                nmx = jnp.where(is_left, split, mx)
                minx[u] = jnp.where(horiz, nmn, minx[u])
                maxx[u] = jnp.where(horiz, nmx, maxx[u])
                miny[u] = jnp.where(horiz, miny[u], nmn)
                maxy[u] = jnp.where(horiz, maxy[u], nmx)
                idx[u] = jnp.where(is_left, lchild, rchild)
        for u in range(UNROLL):
            out_v[pl.ds(pbase0 + u * L, L)] = plsc.load_gather(vals_v, [idx[u]])
        return carry

    lax.fori_loop(0, VECS // UNROLL, body, 0)
    pltpu.sync_copy(out_v, out_hbm.at[pl.ds(base, PTS_PER_W)])


def kernel(grid_points, split_points, split_directions, values,
           left_matrix, right_matrix, max_depth):
    del max_depth
    packed = _pack_tables(left_matrix, right_matrix, split_directions)
    zero16 = jnp.zeros((L,), jnp.int32)
    xs = grid_points[:, 0]
    ys = grid_points[:, 1]
    flat = _traverse(xs, ys, packed, split_points, values, zero16)
    return flat.reshape(NPX, NPY)


# speedup vs baseline: 1.0228x; 1.0228x over previous
import functools

import jax
import jax.numpy as jnp
from jax import lax
from jax.experimental import pallas as pl
from jax.experimental.pallas import tpu as pltpu
from jax.experimental.pallas import tpu_sc as plsc

NVD = 1024
NPX = 1024
NPY = 1024
B = NPX * NPY
NC = 2
NS = 16
L = 16
NW = NC * NS
PTS_PER_W = B // NW
VECS = PTS_PER_W // L
MAX_DEPTH = 16
VERT_BIT = 1 << 20
UNROLL = 4


def _prep_body(lm_ref, rm_ref, sd_ref, packed_ref):
    col = lax.broadcasted_iota(jnp.int32, (NVD, NVD), 1).astype(jnp.float32)
    li = jnp.sum(lm_ref[...] * col, axis=1, keepdims=True).astype(jnp.int32)
    ri = jnp.sum(rm_ref[...] * col, axis=1, keepdims=True).astype(jnp.int32)
    vert = (sd_ref[...] != 0.0).astype(jnp.int32)
    packed_ref[...] = (vert << 20) | (li << 10) | ri


def _pack_tables(left_matrix, right_matrix, split_directions):
    packed = pl.pallas_call(
        _prep_body,
        out_shape=jax.ShapeDtypeStruct((NVD, 1), jnp.int32),
    )(left_matrix, right_matrix, split_directions.reshape(NVD, 1))
    return packed.reshape(NVD)


_mesh = plsc.VectorSubcoreMesh(core_axis_name="c", subcore_axis_name="s")


@functools.partial(
    pl.kernel,
    mesh=_mesh,
    out_type=jax.ShapeDtypeStruct((B,), jnp.float32),
    compiler_params=pltpu.CompilerParams(needs_layout_passes=False),
    scratch_types=[
        pltpu.VMEM((PTS_PER_W,), jnp.float32),
        pltpu.VMEM((PTS_PER_W,), jnp.float32),
        pltpu.VMEM((NVD,), jnp.int32),
        pltpu.VMEM((NVD,), jnp.float32),
        pltpu.VMEM((NVD,), jnp.float32),
        pltpu.VMEM((PTS_PER_W,), jnp.float32),
        pltpu.VMEM((L,), jnp.int32),
    ],
)
def _traverse(xs_hbm, ys_hbm, packed_hbm, sp_hbm, vals_hbm, zero_hbm, out_hbm,
              xs_v, ys_v, packed_v, sp_v, vals_v, out_v, zero_v):
    wid = lax.axis_index("s") * NC + lax.axis_index("c")
    base = wid * PTS_PER_W
    pltpu.sync_copy(xs_hbm.at[pl.ds(base, PTS_PER_W)], xs_v)
    pltpu.sync_copy(ys_hbm.at[pl.ds(base, PTS_PER_W)], ys_v)
    pltpu.sync_copy(packed_hbm, packed_v)
    pltpu.sync_copy(sp_hbm, sp_v)
    pltpu.sync_copy(vals_hbm, vals_v)
    pltpu.sync_copy(zero_hbm, zero_v)

    idx0 = zero_v[...]

    def body(i, carry):
        pbase0 = i * (L * UNROLL)
        x = [None] * UNROLL
        y = [None] * UNROLL
        idx = [idx0] * UNROLL
        minx = [jnp.zeros((L,), jnp.float32)] * UNROLL
        maxx = [jnp.ones((L,), jnp.float32)] * UNROLL
        miny = [jnp.zeros((L,), jnp.float32)] * UNROLL
        maxy = [jnp.ones((L,), jnp.float32)] * UNROLL
        for u in range(UNROLL):
            pb = pbase0 + u * L
            x[u] = xs_v[pl.ds(pb, L)]
            y[u] = ys_v[pl.ds(pb, L)]
        for _ in range(MAX_DEPTH):
            for u in range(UNROLL):
                packed = plsc.load_gather(packed_v, [idx[u]])
                sp = plsc.load_gather(sp_v, [idx[u]])
                horiz = packed < VERT_BIT
                mn = jnp.where(horiz, minx[u], miny[u])
                mx = jnp.where(horiz, maxx[u], maxy[u])
                pos = jnp.where(horiz, x[u], y[u])
                split = mn + sp * (mx - mn)
                is_left = pos < split
                nmn = jnp.where(is_left, mn, split)
                nmx = jnp.where(is_left, split, mx)
                minx[u] = jnp.where(horiz, nmn, minx[u])
                maxx[u] = jnp.where(horiz, nmx, maxx[u])
                miny[u] = jnp.where(horiz, miny[u], nmn)
                maxy[u] = jnp.where(horiz, maxy[u], nmx)
                sh = jnp.where(is_left, 10, 0)
                idx[u] = (packed >> sh) & (NVD - 1)
        for u in range(UNROLL):
            out_v[pl.ds(pbase0 + u * L, L)] = plsc.load_gather(vals_v, [idx[u]])
        return carry

    lax.fori_loop(0, VECS // UNROLL, body, 0)
    pltpu.sync_copy(out_v, out_hbm.at[pl.ds(base, PTS_PER_W)])


def kernel(grid_points, split_points, split_directions, values,
           left_matrix, right_matrix, max_depth):
    del max_depth
    packed = _pack_tables(left_matrix, right_matrix, split_directions)
    zero16 = jnp.zeros((L,), jnp.int32)
    xs = grid_points[:, 0]
    ys = grid_points[:, 1]
    flat = _traverse(xs, ys, packed, split_points, values, zero16)
    return flat.reshape(NPX, NPY)


# speedup vs baseline: 1.0430x; 1.0197x over previous
import functools

import jax
import jax.numpy as jnp
from jax import lax
from jax.experimental import pallas as pl
from jax.experimental.pallas import tpu as pltpu
from jax.experimental.pallas import tpu_sc as plsc

NVD = 1024
NPX = 1024
NPY = 1024
B = NPX * NPY
NC = 2
NS = 16
L = 16
NW = NC * NS
PTS_PER_W = B // NW
VECS = PTS_PER_W // L
MAX_DEPTH = 16
VERT_BIT = 1 << 20
UNROLL = 6


def _prep_body(lm_ref, rm_ref, sd_ref, packed_ref):
    col = lax.broadcasted_iota(jnp.int32, (NVD, NVD), 1).astype(jnp.float32)
    li = jnp.sum(lm_ref[...] * col, axis=1, keepdims=True).astype(jnp.int32)
    ri = jnp.sum(rm_ref[...] * col, axis=1, keepdims=True).astype(jnp.int32)
    vert = (sd_ref[...] != 0.0).astype(jnp.int32)
    packed_ref[...] = (vert << 20) | (li << 10) | ri


def _pack_tables(left_matrix, right_matrix, split_directions):
    packed = pl.pallas_call(
        _prep_body,
        out_shape=jax.ShapeDtypeStruct((NVD, 1), jnp.int32),
    )(left_matrix, right_matrix, split_directions.reshape(NVD, 1))
    return packed.reshape(NVD)


_mesh = plsc.VectorSubcoreMesh(core_axis_name="c", subcore_axis_name="s")


@functools.partial(
    pl.kernel,
    mesh=_mesh,
    out_type=jax.ShapeDtypeStruct((B,), jnp.float32),
    compiler_params=pltpu.CompilerParams(needs_layout_passes=False),
    scratch_types=[
        pltpu.VMEM((PTS_PER_W,), jnp.float32),
        pltpu.VMEM((PTS_PER_W,), jnp.float32),
        pltpu.VMEM((NVD,), jnp.int32),
        pltpu.VMEM((NVD,), jnp.float32),
        pltpu.VMEM((NVD,), jnp.float32),
        pltpu.VMEM((PTS_PER_W,), jnp.float32),
        pltpu.VMEM((L,), jnp.int32),
    ],
)
def _traverse(xs_hbm, ys_hbm, packed_hbm, sp_hbm, vals_hbm, zero_hbm, out_hbm,
              xs_v, ys_v, packed_v, sp_v, vals_v, out_v, zero_v):
    wid = lax.axis_index("s") * NC + lax.axis_index("c")
    base = wid * PTS_PER_W
    pltpu.sync_copy(xs_hbm.at[pl.ds(base, PTS_PER_W)], xs_v)
    pltpu.sync_copy(ys_hbm.at[pl.ds(base, PTS_PER_W)], ys_v)
    pltpu.sync_copy(packed_hbm, packed_v)
    pltpu.sync_copy(sp_hbm, sp_v)
    pltpu.sync_copy(vals_hbm, vals_v)
    pltpu.sync_copy(zero_hbm, zero_v)

    idx0 = zero_v[...]

    def body(i, carry):
        pbase0 = i * (L * UNROLL)
        x = [None] * UNROLL
        y = [None] * UNROLL
        idx = [idx0] * UNROLL
        minx = [jnp.zeros((L,), jnp.float32)] * UNROLL
        maxx = [jnp.ones((L,), jnp.float32)] * UNROLL
        miny = [jnp.zeros((L,), jnp.float32)] * UNROLL
        maxy = [jnp.ones((L,), jnp.float32)] * UNROLL
        for u in range(UNROLL):
            pb = pbase0 + u * L
            x[u] = xs_v[pl.ds(pb, L)]
            y[u] = ys_v[pl.ds(pb, L)]
        for _ in range(MAX_DEPTH):
            for u in range(UNROLL):
                packed = plsc.load_gather(packed_v, [idx[u]])
                sp = plsc.load_gather(sp_v, [idx[u]])
                horiz = packed < VERT_BIT
                lchild = (packed >> 10) & (NVD - 1)
                rchild = packed & (NVD - 1)
                mn = jnp.where(horiz, minx[u], miny[u])
                mx = jnp.where(horiz, maxx[u], maxy[u])
                pos = jnp.where(horiz, x[u], y[u])
                split = mn + sp * (mx - mn)
                is_left = pos < split
                nmn = jnp.where(is_left, mn, split)
                nmx = jnp.where(is_left, split, mx)
                minx[u] = jnp.where(horiz, nmn, minx[u])
                maxx[u] = jnp.where(horiz, nmx, maxx[u])
                miny[u] = jnp.where(horiz, miny[u], nmn)
                maxy[u] = jnp.where(horiz, maxy[u], nmx)
                idx[u] = jnp.where(is_left, lchild, rchild)
        for u in range(UNROLL):
            out_v[pl.ds(pbase0 + u * L, L)] = plsc.load_gather(vals_v, [idx[u]])
        return carry

    lax.fori_loop(0, VECS // UNROLL, body, 0)
    pltpu.sync_copy(out_v, out_hbm.at[pl.ds(base, PTS_PER_W)])


def kernel(grid_points, split_points, split_directions, values,
           left_matrix, right_matrix, max_depth):
    del max_depth
    packed = _pack_tables(left_matrix, right_matrix, split_directions)
    zero16 = jnp.zeros((L,), jnp.int32)
    xs = grid_points[:, 0]
    ys = grid_points[:, 1]
    flat = _traverse(xs, ys, packed, split_points, values, zero16)
    return flat.reshape(NPX, NPY)
